# baseline (speedup 1.0000x reference)
import functools

import jax
import jax.numpy as jnp
from jax import lax
from jax.experimental import pallas as pl
from jax.experimental.pallas import tpu as pltpu
from jax.experimental.pallas import tpu_sc as plsc

NUM_CLASSES = 100000
H_DIM = 128
BATCH = 4096
HALF_BATCH = BATCH // 2

_info = plsc.get_sparse_core_info()
_NS = _info.num_subcores
_B_PER_W = HALF_BATCH // _NS


def _make_gather(tag):
    mesh = plsc.VectorSubcoreMesh(
        core_axis_name="c", subcore_axis_name="s", num_cores=1
    )

    @functools.partial(
        pl.kernel,
        mesh=mesh,
        out_type=jax.ShapeDtypeStruct((HALF_BATCH, H_DIM), jnp.float32),
        scratch_types=[
            pltpu.VMEM((_B_PER_W,), jnp.int32),
            pltpu.VMEM((_B_PER_W, H_DIM), jnp.float32),
            pltpu.SemaphoreType.DMA,
        ],
        name=f"half_gather_{tag}",
    )
    def gather_kernel(table_hbm, idx_hbm, out_hbm, idx_v, rows_v, sem):
        sid = lax.axis_index("s")
        base = sid * _B_PER_W
        pltpu.sync_copy(idx_hbm.at[pl.ds(base, _B_PER_W)], idx_v)
        pltpu.async_copy(table_hbm.at[idx_v], rows_v, sem).wait()
        pltpu.sync_copy(rows_v, out_hbm.at[pl.ds(base, _B_PER_W)])

    return gather_kernel


_gather_a = _make_gather("a")
_gather_b = _make_gather("b")


def kernel(label, table):
    idx = label.astype(jnp.int32)
    a = _gather_a(table, idx[:HALF_BATCH])
    b = _gather_b(table, idx[HALF_BATCH:])
    return jnp.concatenate([a, b], axis=0)


# speedup vs baseline: 1.3923x; 1.3923x over previous
import functools

import jax
import jax.numpy as jnp
from jax import lax
from jax.experimental import pallas as pl
from jax.experimental.pallas import tpu as pltpu
from jax.experimental.pallas import tpu_sc as plsc

NUM_CLASSES = 100000
H_DIM = 128
BATCH = 4096

_info = plsc.get_sparse_core_info()
_NS = _info.num_subcores
_B_PER_W = BATCH // _NS


def _make_gather():
    mesh = plsc.VectorSubcoreMesh(
        core_axis_name="c", subcore_axis_name="s", num_cores=1
    )

    @functools.partial(
        pl.kernel,
        mesh=mesh,
        out_type=jax.ShapeDtypeStruct((BATCH, H_DIM), jnp.float32),
        scratch_types=[
            pltpu.VMEM((_B_PER_W,), jnp.int32),
            pltpu.VMEM((_B_PER_W, H_DIM), jnp.float32),
            pltpu.SemaphoreType.DMA,
        ],
    )
    def gather_kernel(table_hbm, idx_hbm, out_hbm, idx_v, rows_v, sem):
        sid = lax.axis_index("s")
        base = sid * _B_PER_W
        pltpu.sync_copy(idx_hbm.at[pl.ds(base, _B_PER_W)], idx_v)
        pltpu.async_copy(table_hbm.at[idx_v], rows_v, sem).wait()
        pltpu.sync_copy(rows_v, out_hbm.at[pl.ds(base, _B_PER_W)])

    return gather_kernel


_gather = _make_gather()


def kernel(label, table):
    return _gather(table, label.astype(jnp.int32))
